# baseline (speedup 1.0000x reference)
import jax
import jax.numpy as jnp
from jax.experimental import pallas as pl
from jax.experimental.pallas import tpu as pltpu

_NCHUNKS = 2


def _stream_copy(x_hbm, o_hbm, vmem, in_sems, out_sems):
    K = vmem.shape[0]
    rows = K // _NCHUNKS
    ins = []
    outs = []
    for i in range(_NCHUNKS):
        sl = pl.ds(i * rows, rows)
        ins.append(pltpu.make_async_copy(
            x_hbm.at[sl, :], vmem.at[sl, :], in_sems.at[i]))
        outs.append(pltpu.make_async_copy(
            vmem.at[sl, :], o_hbm.at[sl, :], out_sems.at[i]))
    for c in ins:
        c.start()
    for i in range(_NCHUNKS):
        ins[i].wait()
        outs[i].start()
    for c in outs:
        c.wait()


def kernel(embedding_weight):
    K, D = embedding_weight.shape
    return pl.pallas_call(
        _stream_copy,
        in_specs=[pl.BlockSpec(memory_space=pl.ANY)],
        out_specs=pl.BlockSpec(memory_space=pl.ANY),
        out_shape=jax.ShapeDtypeStruct((K, D), embedding_weight.dtype),
        scratch_shapes=[
            pltpu.VMEM((K, D), embedding_weight.dtype),
            pltpu.SemaphoreType.DMA((_NCHUNKS,)),
            pltpu.SemaphoreType.DMA((_NCHUNKS,)),
        ],
    )(embedding_weight)


# speedup vs baseline: 1.0217x; 1.0217x over previous
import jax
import jax.numpy as jnp
from jax.experimental import pallas as pl
from jax.experimental.pallas import tpu as pltpu

_NCHUNKS = 4


def _stream_copy(x_hbm, o_hbm, vmem, in_sems, out_sems):
    K = vmem.shape[0]
    rows = K // _NCHUNKS
    ins = []
    outs = []
    for i in range(_NCHUNKS):
        sl = pl.ds(i * rows, rows)
        ins.append(pltpu.make_async_copy(
            x_hbm.at[sl, :], vmem.at[sl, :], in_sems.at[i]))
        outs.append(pltpu.make_async_copy(
            vmem.at[sl, :], o_hbm.at[sl, :], out_sems.at[i]))
    for c in ins:
        c.start()
    for i in range(_NCHUNKS):
        ins[i].wait()
        outs[i].start()
    for c in outs:
        c.wait()


def kernel(embedding_weight):
    K, D = embedding_weight.shape
    return pl.pallas_call(
        _stream_copy,
        in_specs=[pl.BlockSpec(memory_space=pl.ANY)],
        out_specs=pl.BlockSpec(memory_space=pl.ANY),
        out_shape=jax.ShapeDtypeStruct((K, D), embedding_weight.dtype),
        scratch_shapes=[
            pltpu.VMEM((K, D), embedding_weight.dtype),
            pltpu.SemaphoreType.DMA((_NCHUNKS,)),
            pltpu.SemaphoreType.DMA((_NCHUNKS,)),
        ],
    )(embedding_weight)
